# Initial kernel scaffold; baseline (speedup 1.0000x reference)
import jax
import jax.numpy as jnp
from jax.experimental import pallas as pl


def kernel(x, weights, p, step_num):
    raise NotImplementedError("write your pallas kernel here")


# speedup vs baseline: 55.5371x; 55.5371x over previous
import functools

import jax
import jax.numpy as jnp
from jax.experimental import pallas as pl
from jax.experimental.pallas import tpu as pltpu

EPS = 1e-06
MAX_P = 1.0
NORM_CONST = 256.0
START_GAMMA_MUL = 1.0
DECAY_GAMMA = 1.0 / 1.15

_NL = 512


def _prep_kernel(x_ref, w_ref, coef_ref, gamma_ref, wtb_ref):
    xb = x_ref[0]
    ssq = jnp.sum(xb * xb, keepdims=True)
    gamma_ref[...] = jnp.minimum(
        jnp.sqrt(ssq) * coef_ref[...], EPS)[None]
    w = w_ref[...]
    e = jnp.exp(w - jnp.max(w))
    wt = e * (NORM_CONST / jnp.sum(e))
    bits = jax.lax.bitcast_convert_type(wt, jnp.int32)
    wtb_ref[...] = jax.lax.shift_right_logical(bits, 9)


def _cmpex_pair(xk, ch, row, j, k):
    is_high = (row & j) != 0
    m = is_high ^ ((row & k) != 0)
    xk_dn = jnp.roll(xk, -j, axis=0)
    xk_up = jnp.roll(xk, j, axis=0)
    ch_dn = jnp.roll(ch, -j, axis=0)
    ch_up = jnp.roll(ch, j, axis=0)
    pxk = jnp.where(is_high, xk_up, xk_dn)
    pch = jnp.where(is_high, ch_up, ch_dn)
    sgp = (xk > pxk) | ((xk == pxk) & (ch > pch))
    tp = sgp ^ m
    return jnp.where(tp, pxk, xk), jnp.where(tp, pch, ch)


def _cmpex_word(word, row, j, k):
    is_high = (row & j) != 0
    m = is_high ^ ((row & k) != 0)
    w_dn = jnp.roll(word, -j, axis=0)
    w_up = jnp.roll(word, j, axis=0)
    pw = jnp.where(is_high, w_up, w_dn)
    tp = (word > pw) ^ m
    return jnp.where(tp, pw, word)


def _main_kernel(x_ref, wtb_ref, gamma_ref, p_ref, out_ref):
    xb = x_ref[0]
    c = xb.shape[0]
    xk = -xb
    ch = jax.lax.broadcasted_iota(jnp.int32, xb.shape, 0).astype(jnp.float32)
    row = jax.lax.broadcasted_iota(jnp.int32, (c, 1), 0)

    k = 2
    while k <= c:
        j = k // 2
        while j >= 1:
            xk, ch = _cmpex_pair(xk, ch, row, j, k)
            j //= 2
        k *= 2

    word = jax.lax.shift_left(ch.astype(jnp.int32), 23) | wtb_ref[...]

    k = 2
    while k <= c:
        j = k // 2
        while j >= 1:
            word = _cmpex_word(word, row, j, k)
            j //= 2
        k *= 2

    wt_g = jax.lax.bitcast_convert_type(
        jax.lax.shift_left(word & 0x7FFFFF, 9), jnp.float32)

    gamma = gamma_ref[0]
    expo = (jax.nn.sigmoid(p_ref[...]) * MAX_P - 2.0) * 0.5
    out_ref[0] = wt_g * jnp.exp(expo * jnp.log(xb * xb + gamma))


@functools.partial(jax.jit, static_argnames=())
def kernel(x, weights, p, step_num):
    b, c, h, w = x.shape
    s = h * w
    xr = x.reshape(b, c, s)
    nl = min(_NL, s)

    coef = (START_GAMMA_MUL
            * jnp.power(jnp.float32(DECAY_GAMMA),
                        jnp.asarray(step_num, jnp.float32))).reshape(1, 1)
    w_col = weights.reshape(c, 1)
    p_arr = p.reshape(1, 1).astype(jnp.float32)

    gamma, wtb = pl.pallas_call(
        _prep_kernel,
        grid=(b,),
        in_specs=[
            pl.BlockSpec((1, c, s), lambda i: (i, 0, 0)),
            pl.BlockSpec((c, 1), lambda i: (0, 0)),
            pl.BlockSpec((1, 1), lambda i: (0, 0)),
        ],
        out_specs=[
            pl.BlockSpec((1, 1, 1), lambda i: (i, 0, 0)),
            pl.BlockSpec((c, 1), lambda i: (0, 0)),
        ],
        out_shape=[
            jax.ShapeDtypeStruct((b, 1, 1), jnp.float32),
            jax.ShapeDtypeStruct((c, 1), jnp.int32),
        ],
    )(xr, w_col, coef)

    out = pl.pallas_call(
        _main_kernel,
        grid=(b, s // nl),
        in_specs=[
            pl.BlockSpec((1, c, nl), lambda i, t: (i, 0, t)),
            pl.BlockSpec((c, 1), lambda i, t: (0, 0)),
            pl.BlockSpec((1, 1, 1), lambda i, t: (i, 0, 0)),
            pl.BlockSpec((1, 1), lambda i, t: (0, 0)),
        ],
        out_specs=pl.BlockSpec((1, c, nl), lambda i, t: (i, 0, t)),
        out_shape=jax.ShapeDtypeStruct((b, c, s), jnp.float32),
    )(xr, wtb, gamma, p_arr)

    return out.reshape(b, c, h, w)
